# Initial kernel scaffold; baseline (speedup 1.0000x reference)
import jax
import jax.numpy as jnp
from jax.experimental import pallas as pl


def kernel(x, topo, edge_index, W1, att_src1, att_dst1, b1, W2, att_src2, att_dst2, b2):
    raise NotImplementedError("write your pallas kernel here")


# speedup vs baseline: 52.8273x; 52.8273x over previous
import functools

import jax
import jax.numpy as jnp
from jax import lax
from jax.experimental import pallas as pl
from jax.experimental.pallas import tpu as pltpu
from jax.experimental.pallas import tpu_sc as plsc

N = 10000
E = 320000
BN = 1000
CHUNK = 128
NCK = E // CHUNK // 32


def _tc1_body(x_ref, topo_ref, w1_ref, as_ref, ad_ref, g_ref, adst_ref):
    xb = x_ref[...]
    tb = topo_ref[...]
    h = (jnp.dot(xb, w1_ref[0:128, :], preferred_element_type=jnp.float32)
         + jnp.dot(tb, w1_ref[128:136, :], preferred_element_type=jnp.float32))
    asrc = jnp.dot(h, as_ref[...], preferred_element_type=jnp.float32)
    adst = jnp.dot(h, ad_ref[...], preferred_element_type=jnp.float32)
    z8 = jnp.zeros_like(asrc)
    g_ref[...] = jnp.concatenate([h, asrc, z8], axis=1)
    adst_ref[...] = jnp.concatenate([adst, z8], axis=1)


def _tc1(x, topo, W1, As1, Ad1):
    return pl.pallas_call(
        _tc1_body,
        grid=(N // BN,),
        in_specs=[
            pl.BlockSpec((BN, 128), lambda i: (i, 0)),
            pl.BlockSpec((BN, 8), lambda i: (i, 0)),
            pl.BlockSpec((136, 64), lambda i: (0, 0)),
            pl.BlockSpec((64, 8), lambda i: (0, 0)),
            pl.BlockSpec((64, 8), lambda i: (0, 0)),
        ],
        out_specs=[
            pl.BlockSpec((BN, 80), lambda i: (i, 0)),
            pl.BlockSpec((BN, 16), lambda i: (i, 0)),
        ],
        out_shape=[
            jax.ShapeDtypeStruct((N, 80), jnp.float32),
            jax.ShapeDtypeStruct((N, 16), jnp.float32),
        ],
    )(x, topo, W1, As1, Ad1)


def _tc2_body(ua_ref, ub_ref, rep_ref, b1_ref, w2_ref, as2_ref, ad2_ref,
              g2_ref, a2_ref):
    u = ua_ref[...] + ub_ref[...]
    s = u[:, 64:72]
    r = 1.0 / (s + 1e-16)
    rex = jnp.dot(r, rep_ref[...], preferred_element_type=jnp.float32)
    z = u[:, 0:64] * rex + b1_ref[...]
    z = jnp.where(z > 0, z, jnp.exp(z) - 1.0)
    h2 = jnp.dot(z, w2_ref[...], preferred_element_type=jnp.float32)
    asrc2 = jnp.dot(h2, as2_ref[...], preferred_element_type=jnp.float32)
    adst2 = jnp.dot(h2, ad2_ref[...], preferred_element_type=jnp.float32)
    g2_ref[...] = jnp.concatenate([h2, asrc2], axis=1)
    a2_ref[...] = adst2


def _tc2(Ua, Ub, Rep8, b1r, W2, As2p, Ad2p):
    return pl.pallas_call(
        _tc2_body,
        grid=(N // BN,),
        in_specs=[
            pl.BlockSpec((BN, 80), lambda i: (i, 0)),
            pl.BlockSpec((BN, 80), lambda i: (i, 0)),
            pl.BlockSpec((8, 64), lambda i: (0, 0)),
            pl.BlockSpec((1, 64), lambda i: (0, 0)),
            pl.BlockSpec((64, 16), lambda i: (0, 0)),
            pl.BlockSpec((16, 16), lambda i: (0, 0)),
            pl.BlockSpec((16, 16), lambda i: (0, 0)),
        ],
        out_specs=[
            pl.BlockSpec((BN, 32), lambda i: (i, 0)),
            pl.BlockSpec((BN, 16), lambda i: (i, 0)),
        ],
        out_shape=[
            jax.ShapeDtypeStruct((N, 32), jnp.float32),
            jax.ShapeDtypeStruct((N, 16), jnp.float32),
        ],
    )(Ua, Ub, Rep8, b1r, W2, As2p, Ad2p)


def _tc3_body(ua_ref, ub_ref, b2_ref, out_ref):
    u = ua_ref[...] + ub_ref[...]
    s = u[:, 16:17]
    o = u[:, 0:16] / (s + 1e-16) + b2_ref[...]
    m = jnp.max(o, axis=1, keepdims=True)
    ex = jnp.exp(o - m)
    lse = jnp.log(jnp.sum(ex, axis=1, keepdims=True))
    out_ref[...] = o - m - lse


def _tc3(Ua, Ub, b2r):
    return pl.pallas_call(
        _tc3_body,
        grid=(N // BN,),
        in_specs=[
            pl.BlockSpec((BN, 32), lambda i: (i, 0)),
            pl.BlockSpec((BN, 32), lambda i: (i, 0)),
            pl.BlockSpec((1, 16), lambda i: (0, 0)),
        ],
        out_specs=pl.BlockSpec((BN, 16), lambda i: (i, 0)),
        out_shape=jax.ShapeDtypeStruct((N, 16), jnp.float32),
    )(Ua, Ub, b2r)


def _take16(v, idx16):
    return lax.gather(
        v, idx16.reshape(16, 1),
        lax.GatherDimensionNumbers(offset_dims=(), collapsed_slice_dims=(0,),
                                   start_index_map=(0,)),
        (1,), mode=lax.GatherScatterMode.PROMISE_IN_BOUNDS)

def _make_edge_pass(DG, HD, NH):
    mesh = plsc.VectorSubcoreMesh(core_axis_name="c", subcore_axis_name="s",
                                  num_cores=2)
    ROWS = 624
    TAIL = N - 16 * ROWS

    @functools.partial(
        pl.kernel,
        mesh=mesh,
        compiler_params=pltpu.CompilerParams(use_tc_tiling_on_sc=False),
        out_type=jax.ShapeDtypeStruct((2, N, DG), jnp.float32),
        scratch_types=[
            pltpu.VMEM((CHUNK,), jnp.int32),
            pltpu.VMEM((CHUNK,), jnp.int32),
            pltpu.VMEM((CHUNK, DG), jnp.float32),
            pltpu.VMEM((CHUNK, 16), jnp.float32),
            pltpu.VMEM((CHUNK, DG), jnp.float32),
            pltpu.VMEM_SHARED((N, DG), jnp.float32),
            pltpu.SemaphoreType.DMA,
        ],
    )
    def edge_pass(g_hbm, src_hbm, dst_hbm, adst_hbm, out_hbm,
                  srcv, dstv, grows, arows, msg, acc, sem):
        cid = lax.axis_index("c")
        sid = lax.axis_index("s")
        wid = sid * 2 + cid
        lane = lax.iota(jnp.int32, 16)
        zero16 = jnp.zeros((16,), jnp.float32)
        headmask = jnp.where(lane < NH, 1.0, 0.0)
        hd_shift = HD.bit_length() - 1
        hidx = [lax.shift_right_logical(lane + 16 * q, hd_shift)
                for q in range(DG // 16 - 1)]

        def _zrow(r, c):
            for q in range(DG // 16):
                msg[r, pl.ds(16 * q, 16)] = zero16
            return c
        lax.fori_loop(0, CHUNK, _zrow, 0)
        rbase = pl.multiple_of(sid * ROWS, 8)
        for j in range(ROWS // CHUNK):
            pltpu.sync_copy(msg, acc.at[pl.ds(rbase + j * CHUNK, CHUNK)])
        rem = ROWS % CHUNK
        if rem:
            pltpu.sync_copy(msg.at[pl.ds(0, rem)],
                            acc.at[pl.ds(rbase + ROWS - rem, rem)])

        @pl.when(sid == 15)
        def _zero_tail():
            pltpu.sync_copy(msg.at[pl.ds(0, TAIL)],
                            acc.at[pl.ds(16 * ROWS, TAIL)])
        plsc.subcore_barrier()

        def _chunk(k, c):
            base_e = pl.multiple_of((wid + 32 * k) * CHUNK, 8)
            pltpu.sync_copy(src_hbm.at[pl.ds(base_e, CHUNK)], srcv)
            pltpu.sync_copy(dst_hbm.at[pl.ds(base_e, CHUNK)], dstv)
            pltpu.async_copy(g_hbm.at[srcv], grows, sem).wait()
            pltpu.async_copy(adst_hbm.at[dstv], arows, sem).wait()

            def _edge(e, c2):
                asrc = grows[e, pl.ds(DG - 16, 16)]
                adst = arows[e, pl.ds(0, 16)]
                al = asrc + adst
                al = jnp.maximum(al, 0.2 * al)
                ev = jnp.exp(al)
                for q in range(DG // 16 - 1):
                    hq = grows[e, pl.ds(16 * q, 16)]
                    eq = _take16(ev, hidx[q])
                    msg[e, pl.ds(16 * q, 16)] = hq * eq
                msg[e, pl.ds(DG - 16, 16)] = ev * headmask
                return c2
            lax.fori_loop(0, CHUNK, _edge, 0)
            pltpu.sync_copy(msg, acc.at[dstv], add=True)
            return c

        nck = jnp.where(wid < (E // CHUNK) % 32, NCK + 1, NCK)
        lax.fori_loop(0, nck, _chunk, 0)
        plsc.subcore_barrier()

        pltpu.sync_copy(acc.at[pl.ds(rbase, ROWS)],
                        out_hbm.at[cid, pl.ds(rbase, ROWS)])

        @pl.when(sid == 15)
        def _write_tail():
            pltpu.sync_copy(acc.at[pl.ds(16 * ROWS, TAIL)],
                            out_hbm.at[cid, pl.ds(16 * ROWS, TAIL)])

    return edge_pass


_EDGE_PASS_CACHE = {}


def _edge_pass(DG, HD, NH):
    key = (DG, HD, NH)
    if key not in _EDGE_PASS_CACHE:
        _EDGE_PASS_CACHE[key] = _make_edge_pass(DG, HD, NH)
    return _EDGE_PASS_CACHE[key]


def kernel(x, topo, edge_index, W1, att_src1, att_dst1, b1,
           W2, att_src2, att_dst2, b2):
    src = edge_index[0]
    dst = edge_index[1]

    eye8 = jnp.eye(8, dtype=jnp.float32)
    As1 = (att_src1.reshape(8, 8)[:, :, None] * eye8[:, None, :]).reshape(64, 8)
    Ad1 = (att_dst1.reshape(8, 8)[:, :, None] * eye8[:, None, :]).reshape(64, 8)
    Rep8 = jnp.repeat(eye8, 8, axis=1)
    z15 = jnp.zeros((16, 15), dtype=jnp.float32)
    As2p = jnp.concatenate([att_src2.reshape(16, 1), z15], axis=1)
    Ad2p = jnp.concatenate([att_dst2.reshape(16, 1), z15], axis=1)

    G1, A1 = _tc1(x, topo, W1, As1, Ad1)
    U1 = _edge_pass(80, 8, 8)(G1, src, dst, A1)
    G2, A2 = _tc2(U1[0], U1[1], Rep8, b1.reshape(1, 64), W2, As2p, Ad2p)
    U2 = _edge_pass(32, 16, 1)(G2, src, dst, A2)
    return _tc3(U2[0], U2[1], b2.reshape(1, 16))


# speedup vs baseline: 175.2166x; 3.3168x over previous
import functools

import jax
import jax.numpy as jnp
from jax import lax
from jax.experimental import pallas as pl
from jax.experimental.pallas import tpu as pltpu
from jax.experimental.pallas import tpu_sc as plsc

N = 10000
E = 320000
BN = 1000
CHUNK = 128
NCK = E // CHUNK // 32


def _tc1_body(x_ref, topo_ref, w1_ref, as_ref, ad_ref, g_ref, adst_ref):
    xb = x_ref[...]
    tb = topo_ref[...]
    h = (jnp.dot(xb, w1_ref[0:128, :], preferred_element_type=jnp.float32)
         + jnp.dot(tb, w1_ref[128:136, :], preferred_element_type=jnp.float32))
    asrc = jnp.dot(h, as_ref[...], preferred_element_type=jnp.float32)
    adst = jnp.dot(h, ad_ref[...], preferred_element_type=jnp.float32)
    z8 = jnp.zeros_like(asrc)
    g_ref[...] = jnp.concatenate([h, asrc, z8], axis=1)
    adst_ref[...] = jnp.concatenate([adst, z8], axis=1)


def _tc1(x, topo, W1, As1, Ad1):
    return pl.pallas_call(
        _tc1_body,
        grid=(N // BN,),
        in_specs=[
            pl.BlockSpec((BN, 128), lambda i: (i, 0)),
            pl.BlockSpec((BN, 8), lambda i: (i, 0)),
            pl.BlockSpec((136, 64), lambda i: (0, 0)),
            pl.BlockSpec((64, 8), lambda i: (0, 0)),
            pl.BlockSpec((64, 8), lambda i: (0, 0)),
        ],
        out_specs=[
            pl.BlockSpec((BN, 80), lambda i: (i, 0)),
            pl.BlockSpec((BN, 16), lambda i: (i, 0)),
        ],
        out_shape=[
            jax.ShapeDtypeStruct((N, 80), jnp.float32),
            jax.ShapeDtypeStruct((N, 16), jnp.float32),
        ],
    )(x, topo, W1, As1, Ad1)


def _tc2_body(ua_ref, ub_ref, rep_ref, b1_ref, w2_ref, as2_ref, ad2_ref,
              g2_ref, a2_ref):
    u = ua_ref[...] + ub_ref[...]
    s = u[:, 64:72]
    r = 1.0 / (s + 1e-16)
    rex = jnp.dot(r, rep_ref[...], preferred_element_type=jnp.float32)
    z = u[:, 0:64] * rex + b1_ref[...]
    z = jnp.where(z > 0, z, jnp.exp(z) - 1.0)
    h2 = jnp.dot(z, w2_ref[...], preferred_element_type=jnp.float32)
    asrc2 = jnp.dot(h2, as2_ref[...], preferred_element_type=jnp.float32)
    adst2 = jnp.dot(h2, ad2_ref[...], preferred_element_type=jnp.float32)
    g2_ref[...] = jnp.concatenate([h2, asrc2], axis=1)
    a2_ref[...] = adst2


def _tc2(Ua, Ub, Rep8, b1r, W2, As2p, Ad2p):
    return pl.pallas_call(
        _tc2_body,
        grid=(N // BN,),
        in_specs=[
            pl.BlockSpec((BN, 80), lambda i: (i, 0)),
            pl.BlockSpec((BN, 80), lambda i: (i, 0)),
            pl.BlockSpec((8, 64), lambda i: (0, 0)),
            pl.BlockSpec((1, 64), lambda i: (0, 0)),
            pl.BlockSpec((64, 16), lambda i: (0, 0)),
            pl.BlockSpec((16, 16), lambda i: (0, 0)),
            pl.BlockSpec((16, 16), lambda i: (0, 0)),
        ],
        out_specs=[
            pl.BlockSpec((BN, 32), lambda i: (i, 0)),
            pl.BlockSpec((BN, 16), lambda i: (i, 0)),
        ],
        out_shape=[
            jax.ShapeDtypeStruct((N, 32), jnp.float32),
            jax.ShapeDtypeStruct((N, 16), jnp.float32),
        ],
    )(Ua, Ub, Rep8, b1r, W2, As2p, Ad2p)


def _tc3_body(ua_ref, ub_ref, b2_ref, out_ref):
    u = ua_ref[...] + ub_ref[...]
    s = u[:, 16:17]
    o = u[:, 0:16] / (s + 1e-16) + b2_ref[...]
    m = jnp.max(o, axis=1, keepdims=True)
    ex = jnp.exp(o - m)
    lse = jnp.log(jnp.sum(ex, axis=1, keepdims=True))
    out_ref[...] = o - m - lse


def _tc3(Ua, Ub, b2r):
    return pl.pallas_call(
        _tc3_body,
        grid=(N // BN,),
        in_specs=[
            pl.BlockSpec((BN, 32), lambda i: (i, 0)),
            pl.BlockSpec((BN, 32), lambda i: (i, 0)),
            pl.BlockSpec((1, 16), lambda i: (0, 0)),
        ],
        out_specs=pl.BlockSpec((BN, 16), lambda i: (i, 0)),
        out_shape=jax.ShapeDtypeStruct((N, 16), jnp.float32),
    )(Ua, Ub, b2r)


def _take16(v, idx16):
    return lax.gather(
        v, idx16.reshape(16, 1),
        lax.GatherDimensionNumbers(offset_dims=(), collapsed_slice_dims=(0,),
                                   start_index_map=(0,)),
        (1,), mode=lax.GatherScatterMode.PROMISE_IN_BOUNDS)

def _make_edge_pass(DG, HD, NH):
    mesh = plsc.VectorSubcoreMesh(core_axis_name="c", subcore_axis_name="s",
                                  num_cores=2)
    ROWS = 624
    TAIL = N - 16 * ROWS
    PER_W = E // 32
    NF = PER_W // CHUNK
    ET = PER_W - NF * CHUNK

    @functools.partial(
        pl.kernel,
        mesh=mesh,
        compiler_params=pltpu.CompilerParams(use_tc_tiling_on_sc=False),
        out_type=jax.ShapeDtypeStruct((2, N, DG), jnp.float32),
        scratch_types=[
            pltpu.VMEM((PER_W,), jnp.int32),
            pltpu.VMEM((PER_W,), jnp.int32),
            pltpu.VMEM((2, CHUNK), jnp.int32),
            pltpu.VMEM((2, CHUNK, DG), jnp.float32),
            pltpu.VMEM((2, CHUNK, 16), jnp.float32),
            pltpu.VMEM((2, CHUNK, DG), jnp.float32),
            pltpu.VMEM((ET,), jnp.int32),
            pltpu.VMEM((ET,), jnp.int32),
            pltpu.VMEM((ET, DG), jnp.float32),
            pltpu.VMEM((ET, 16), jnp.float32),
            pltpu.VMEM((ET, DG), jnp.float32),
            pltpu.VMEM_SHARED((N, DG), jnp.float32),
            pltpu.SemaphoreType.DMA,
            pltpu.SemaphoreType.DMA,
            pltpu.SemaphoreType.DMA,
            pltpu.SemaphoreType.DMA,
        ],
    )
    def edge_pass(g_hbm, src_hbm, dst_hbm, adst_hbm, out_hbm,
                  srcall, dstall, dsts, grows, arows, msg,
                  srct, dstt, growt, arowt, msgt, acc,
                  semg0, semg1, sems0, sems1):
        cid = lax.axis_index("c")
        sid = lax.axis_index("s")
        wid = sid * 2 + cid
        lane = lax.iota(jnp.int32, 16)
        zero16 = jnp.zeros((16,), jnp.float32)
        headmask = jnp.where(lane < NH, 1.0, 0.0)
        hd_shift = HD.bit_length() - 1
        hidx = [lax.shift_right_logical(lane + 16 * q, hd_shift)
                for q in range(DG // 16 - 1)]
        semg = (semg0, semg1)
        sems = (sems0, sems1)

        ebase = pl.multiple_of(wid * PER_W, 8)
        pltpu.sync_copy(src_hbm.at[pl.ds(ebase, PER_W)], srcall)
        pltpu.sync_copy(dst_hbm.at[pl.ds(ebase, PER_W)], dstall)

        def fetch(k, b):
            off = pl.multiple_of(k * CHUNK, 8)
            pltpu.async_copy(g_hbm.at[srcall.at[pl.ds(off, CHUNK)]],
                             grows.at[b], semg[b])
            pltpu.async_copy(adst_hbm.at[dstall.at[pl.ds(off, CHUNK)]],
                             arows.at[b], semg[b])

        def wait_fetch(k, b):
            off = pl.multiple_of(k * CHUNK, 8)
            pltpu.make_async_copy(g_hbm.at[srcall.at[pl.ds(off, CHUNK)]],
                                  grows.at[b], semg[b]).wait()
            pltpu.make_async_copy(adst_hbm.at[dstall.at[pl.ds(off, CHUNK)]],
                                  arows.at[b], semg[b]).wait()

        def fire_scatter(b):
            pltpu.async_copy(msg.at[b], acc.at[dsts.at[b]], sems[b], add=True)

        def drain_scatter(b):
            pltpu.make_async_copy(msg.at[b], acc.at[dsts.at[b]],
                                  sems[b]).wait()

        def compute(k, b):
            off = pl.multiple_of(k * CHUNK, 8)

            @plsc.parallel_loop(0, CHUNK, step=16)
            def _cp(i):
                dsts[b, pl.ds(i, 16)] = dstall[pl.ds(off + i, 16)]

            @plsc.parallel_loop(0, CHUNK, unroll=2)
            def _edge(e):
                asrc = grows[b, e, pl.ds(DG - 16, 16)]
                adst = arows[b, e, pl.ds(0, 16)]
                al = asrc + adst
                al = jnp.maximum(al, 0.2 * al)
                ev = jnp.exp(al)
                for q in range(DG // 16 - 1):
                    hq = grows[b, e, pl.ds(16 * q, 16)]
                    eq = _take16(ev, hidx[q])
                    msg[b, e, pl.ds(16 * q, 16)] = hq * eq
                msg[b, e, pl.ds(DG - 16, 16)] = ev * headmask

        fetch(0, 0)
        fetch(1, 1)

        def _zrow(r, c):
            for q in range(DG // 16):
                msg[0, r, pl.ds(16 * q, 16)] = zero16
            return c
        lax.fori_loop(0, CHUNK, _zrow, 0)
        rbase = pl.multiple_of(sid * ROWS, 8)
        for j in range(ROWS // CHUNK):
            pltpu.sync_copy(msg.at[0], acc.at[pl.ds(rbase + j * CHUNK, CHUNK)])
        rem = ROWS % CHUNK
        if rem:
            pltpu.sync_copy(msg.at[0, pl.ds(0, rem)],
                            acc.at[pl.ds(rbase + ROWS - rem, rem)])

        @pl.when(sid == 15)
        def _zero_tail():
            pltpu.sync_copy(msg.at[0, pl.ds(0, TAIL)],
                            acc.at[pl.ds(16 * ROWS, TAIL)])
        plsc.subcore_barrier()

        def _outer(j, c):
            @pl.when(j > 0)
            def _drain_prev():
                drain_scatter(0)
                drain_scatter(1)
            for b in range(2):
                k = 2 * j + b
                wait_fetch(k, b)
                compute(k, b)
                fire_scatter(b)

                @pl.when(k + 2 < NF)
                def _prefetch():
                    fetch(k + 2, b)
            return c
        lax.fori_loop(0, NF // 2, _outer, 0)
        drain_scatter(0)
        drain_scatter(1)

        toff = NF * CHUNK
        srct[pl.ds(0, ET)] = srcall[pl.ds(toff, ET)]
        dstt[pl.ds(0, ET)] = dstall[pl.ds(toff, ET)]
        pltpu.async_copy(g_hbm.at[srct], growt, semg0).wait()
        pltpu.async_copy(adst_hbm.at[dstt], arowt, semg0).wait()

        @plsc.parallel_loop(0, ET, unroll=2)
        def _tedge(e):
            asrc = growt[e, pl.ds(DG - 16, 16)]
            adst = arowt[e, pl.ds(0, 16)]
            al = asrc + adst
            al = jnp.maximum(al, 0.2 * al)
            ev = jnp.exp(al)
            for q in range(DG // 16 - 1):
                hq = growt[e, pl.ds(16 * q, 16)]
                eq = _take16(ev, hidx[q])
                msgt[e, pl.ds(16 * q, 16)] = hq * eq
            msgt[e, pl.ds(DG - 16, 16)] = ev * headmask

        pltpu.sync_copy(msgt, acc.at[dstt], add=True)
        plsc.subcore_barrier()

        pltpu.sync_copy(acc.at[pl.ds(rbase, ROWS)],
                        out_hbm.at[cid, pl.ds(rbase, ROWS)])

        @pl.when(sid == 15)
        def _write_tail():
            pltpu.sync_copy(acc.at[pl.ds(16 * ROWS, TAIL)],
                            out_hbm.at[cid, pl.ds(16 * ROWS, TAIL)])

    return edge_pass


_EDGE_PASS_CACHE = {}


def _edge_pass(DG, HD, NH):
    key = (DG, HD, NH)
    if key not in _EDGE_PASS_CACHE:
        _EDGE_PASS_CACHE[key] = _make_edge_pass(DG, HD, NH)
    return _EDGE_PASS_CACHE[key]


def kernel(x, topo, edge_index, W1, att_src1, att_dst1, b1,
           W2, att_src2, att_dst2, b2):
    src = edge_index[0]
    dst = edge_index[1]

    eye8 = jnp.eye(8, dtype=jnp.float32)
    As1 = (att_src1.reshape(8, 8)[:, :, None] * eye8[:, None, :]).reshape(64, 8)
    Ad1 = (att_dst1.reshape(8, 8)[:, :, None] * eye8[:, None, :]).reshape(64, 8)
    Rep8 = jnp.repeat(eye8, 8, axis=1)
    z15 = jnp.zeros((16, 15), dtype=jnp.float32)
    As2p = jnp.concatenate([att_src2.reshape(16, 1), z15], axis=1)
    Ad2p = jnp.concatenate([att_dst2.reshape(16, 1), z15], axis=1)

    G1, A1 = _tc1(x, topo, W1, As1, Ad1)
    U1 = _edge_pass(80, 8, 8)(G1, src, dst, A1)
    G2, A2 = _tc2(U1[0], U1[1], Rep8, b1.reshape(1, 64), W2, As2p, Ad2p)
    U2 = _edge_pass(32, 16, 1)(G2, src, dst, A2)
    return _tc3(U2[0], U2[1], b2.reshape(1, 16))


# speedup vs baseline: 191.4869x; 1.0929x over previous
import functools

import jax
import jax.numpy as jnp
from jax import lax
from jax.experimental import pallas as pl
from jax.experimental.pallas import tpu as pltpu
from jax.experimental.pallas import tpu_sc as plsc

N = 10000
E = 320000
BN = 1000
CHUNK = 128
NCK = E // CHUNK // 32


def _tc1_body(x_ref, topo_ref, w1_ref, as_ref, ad_ref, g_ref, adst_ref):
    xb = x_ref[...]
    tb = topo_ref[...]
    h = (jnp.dot(xb, w1_ref[0:128, :], preferred_element_type=jnp.float32)
         + jnp.dot(tb, w1_ref[128:136, :], preferred_element_type=jnp.float32))
    asrc = jnp.dot(h, as_ref[...], preferred_element_type=jnp.float32)
    adst = jnp.dot(h, ad_ref[...], preferred_element_type=jnp.float32)
    z8 = jnp.zeros_like(asrc)
    g_ref[...] = jnp.concatenate([h, asrc, z8], axis=1)
    adst_ref[...] = jnp.concatenate([adst, z8], axis=1)


def _tc1(x, topo, W1, As1, Ad1):
    return pl.pallas_call(
        _tc1_body,
        grid=(N // BN,),
        in_specs=[
            pl.BlockSpec((BN, 128), lambda i: (i, 0)),
            pl.BlockSpec((BN, 8), lambda i: (i, 0)),
            pl.BlockSpec((136, 64), lambda i: (0, 0)),
            pl.BlockSpec((64, 8), lambda i: (0, 0)),
            pl.BlockSpec((64, 8), lambda i: (0, 0)),
        ],
        out_specs=[
            pl.BlockSpec((BN, 80), lambda i: (i, 0)),
            pl.BlockSpec((BN, 16), lambda i: (i, 0)),
        ],
        out_shape=[
            jax.ShapeDtypeStruct((N, 80), jnp.float32),
            jax.ShapeDtypeStruct((N, 16), jnp.float32),
        ],
    )(x, topo, W1, As1, Ad1)


def _tc2_body(u_ref, rep_ref, b1_ref, w2_ref, as2_ref, ad2_ref,
              g2_ref, a2_ref):
    u = u_ref[0] + u_ref[1]
    s = u[:, 64:72]
    r = 1.0 / (s + 1e-16)
    rex = jnp.dot(r, rep_ref[...], preferred_element_type=jnp.float32)
    z = u[:, 0:64] * rex + b1_ref[...]
    z = jnp.where(z > 0, z, jnp.exp(z) - 1.0)
    h2 = jnp.dot(z, w2_ref[...], preferred_element_type=jnp.float32)
    asrc2 = jnp.dot(h2, as2_ref[...], preferred_element_type=jnp.float32)
    adst2 = jnp.dot(h2, ad2_ref[...], preferred_element_type=jnp.float32)
    g2_ref[...] = jnp.concatenate([h2, asrc2], axis=1)
    a2_ref[...] = adst2


def _tc2(U, Rep8, b1r, W2, As2p, Ad2p):
    return pl.pallas_call(
        _tc2_body,
        grid=(N // BN,),
        in_specs=[
            pl.BlockSpec((2, BN, 80), lambda i: (0, i, 0)),
            pl.BlockSpec((8, 64), lambda i: (0, 0)),
            pl.BlockSpec((1, 64), lambda i: (0, 0)),
            pl.BlockSpec((64, 16), lambda i: (0, 0)),
            pl.BlockSpec((16, 16), lambda i: (0, 0)),
            pl.BlockSpec((16, 16), lambda i: (0, 0)),
        ],
        out_specs=[
            pl.BlockSpec((BN, 32), lambda i: (i, 0)),
            pl.BlockSpec((BN, 16), lambda i: (i, 0)),
        ],
        out_shape=[
            jax.ShapeDtypeStruct((N, 32), jnp.float32),
            jax.ShapeDtypeStruct((N, 16), jnp.float32),
        ],
    )(U, Rep8, b1r, W2, As2p, Ad2p)


def _tc3_body(u_ref, b2_ref, out_ref):
    u = u_ref[0] + u_ref[1]
    s = u[:, 16:17]
    o = u[:, 0:16] / (s + 1e-16) + b2_ref[...]
    m = jnp.max(o, axis=1, keepdims=True)
    ex = jnp.exp(o - m)
    lse = jnp.log(jnp.sum(ex, axis=1, keepdims=True))
    out_ref[...] = o - m - lse


def _tc3(U, b2r):
    return pl.pallas_call(
        _tc3_body,
        grid=(N // BN,),
        in_specs=[
            pl.BlockSpec((2, BN, 32), lambda i: (0, i, 0)),
            pl.BlockSpec((1, 16), lambda i: (0, 0)),
        ],
        out_specs=pl.BlockSpec((BN, 16), lambda i: (i, 0)),
        out_shape=jax.ShapeDtypeStruct((N, 16), jnp.float32),
    )(U, b2r)


def _take16(v, idx16):
    return lax.gather(
        v, idx16.reshape(16, 1),
        lax.GatherDimensionNumbers(offset_dims=(), collapsed_slice_dims=(0,),
                                   start_index_map=(0,)),
        (1,), mode=lax.GatherScatterMode.PROMISE_IN_BOUNDS)

def _make_edge_pass(DG, HD, NH):
    mesh = plsc.VectorSubcoreMesh(core_axis_name="c", subcore_axis_name="s",
                                  num_cores=2)
    ROWS = 624
    TAIL = N - 16 * ROWS
    PER_W = E // 32
    NF = PER_W // CHUNK
    ET = PER_W - NF * CHUNK

    @functools.partial(
        pl.kernel,
        mesh=mesh,
        compiler_params=pltpu.CompilerParams(use_tc_tiling_on_sc=False),
        out_type=jax.ShapeDtypeStruct((2, N, DG), jnp.float32),
        scratch_types=[
            pltpu.VMEM((PER_W,), jnp.int32),
            pltpu.VMEM((PER_W,), jnp.int32),
            pltpu.VMEM((2, CHUNK), jnp.int32),
            pltpu.VMEM((2, CHUNK, DG), jnp.float32),
            pltpu.VMEM((2, CHUNK, 16), jnp.float32),
            pltpu.VMEM((2, CHUNK, DG), jnp.float32),
            pltpu.VMEM((ET,), jnp.int32),
            pltpu.VMEM((ET,), jnp.int32),
            pltpu.VMEM((ET, DG), jnp.float32),
            pltpu.VMEM((ET, 16), jnp.float32),
            pltpu.VMEM((ET, DG), jnp.float32),
            pltpu.VMEM_SHARED((N, DG), jnp.float32),
            pltpu.SemaphoreType.DMA,
            pltpu.SemaphoreType.DMA,
            pltpu.SemaphoreType.DMA,
            pltpu.SemaphoreType.DMA,
        ],
    )
    def edge_pass(g_hbm, ei_hbm, adst_hbm, out_hbm,
                  srcall, dstall, dsts, grows, arows, msg,
                  srct, dstt, growt, arowt, msgt, acc,
                  semg0, semg1, sems0, sems1):
        cid = lax.axis_index("c")
        sid = lax.axis_index("s")
        wid = sid * 2 + cid
        lane = lax.iota(jnp.int32, 16)
        zero16 = jnp.zeros((16,), jnp.float32)
        headmask = jnp.where(lane < NH, 1.0, 0.0)
        hd_shift = HD.bit_length() - 1
        hidx = [lax.shift_right_logical(lane + 16 * q, hd_shift)
                for q in range(DG // 16 - 1)]
        semg = (semg0, semg1)
        sems = (sems0, sems1)

        ebase = pl.multiple_of(wid * PER_W, 8)
        pltpu.sync_copy(ei_hbm.at[0, pl.ds(ebase, PER_W)], srcall)
        pltpu.sync_copy(ei_hbm.at[1, pl.ds(ebase, PER_W)], dstall)

        def fetch(k, b):
            off = pl.multiple_of(k * CHUNK, 8)
            pltpu.async_copy(g_hbm.at[srcall.at[pl.ds(off, CHUNK)]],
                             grows.at[b], semg[b])
            pltpu.async_copy(adst_hbm.at[dstall.at[pl.ds(off, CHUNK)]],
                             arows.at[b], semg[b])

        def wait_fetch(k, b):
            off = pl.multiple_of(k * CHUNK, 8)
            pltpu.make_async_copy(g_hbm.at[srcall.at[pl.ds(off, CHUNK)]],
                                  grows.at[b], semg[b]).wait()
            pltpu.make_async_copy(adst_hbm.at[dstall.at[pl.ds(off, CHUNK)]],
                                  arows.at[b], semg[b]).wait()

        def fire_scatter(b):
            pltpu.async_copy(msg.at[b], acc.at[dsts.at[b]], sems[b], add=True)

        def drain_scatter(b):
            pltpu.make_async_copy(msg.at[b], acc.at[dsts.at[b]],
                                  sems[b]).wait()

        def compute(k, b):
            off = pl.multiple_of(k * CHUNK, 8)

            for i in range(0, CHUNK, 16):
                dsts[b, pl.ds(i, 16)] = dstall[pl.ds(off + i, 16)]

            @plsc.parallel_loop(0, CHUNK, unroll=4)
            def _edge(e):
                asrc = grows[b, e, pl.ds(DG - 16, 16)]
                adst = arows[b, e, pl.ds(0, 16)]
                al = asrc + adst
                al = jnp.maximum(al, 0.2 * al)
                ev = jnp.exp(al)
                for q in range(DG // 16 - 1):
                    hq = grows[b, e, pl.ds(16 * q, 16)]
                    eq = _take16(ev, hidx[q])
                    msg[b, e, pl.ds(16 * q, 16)] = hq * eq
                msg[b, e, pl.ds(DG - 16, 16)] = ev * headmask

        fetch(0, 0)
        fetch(1, 1)

        def _zrow(r, c):
            for q in range(DG // 16):
                msg[0, r, pl.ds(16 * q, 16)] = zero16
            return c
        lax.fori_loop(0, CHUNK, _zrow, 0)
        rbase = pl.multiple_of(sid * ROWS, 8)
        for j in range(ROWS // CHUNK):
            pltpu.sync_copy(msg.at[0], acc.at[pl.ds(rbase + j * CHUNK, CHUNK)])
        rem = ROWS % CHUNK
        if rem:
            pltpu.sync_copy(msg.at[0, pl.ds(0, rem)],
                            acc.at[pl.ds(rbase + ROWS - rem, rem)])

        @pl.when(sid == 15)
        def _zero_tail():
            pltpu.sync_copy(msg.at[0, pl.ds(0, TAIL)],
                            acc.at[pl.ds(16 * ROWS, TAIL)])
        plsc.subcore_barrier()

        def _outer(j, c):
            @pl.when(j > 0)
            def _drain_prev():
                drain_scatter(0)
                drain_scatter(1)
            for b in range(2):
                k = 2 * j + b
                wait_fetch(k, b)
                compute(k, b)
                fire_scatter(b)

                @pl.when(k + 2 < NF)
                def _prefetch():
                    fetch(k + 2, b)
            return c
        lax.fori_loop(0, NF // 2, _outer, 0)
        drain_scatter(0)
        drain_scatter(1)

        toff = NF * CHUNK
        srct[pl.ds(0, ET)] = srcall[pl.ds(toff, ET)]
        dstt[pl.ds(0, ET)] = dstall[pl.ds(toff, ET)]
        pltpu.async_copy(g_hbm.at[srct], growt, semg0).wait()
        pltpu.async_copy(adst_hbm.at[dstt], arowt, semg0).wait()

        @plsc.parallel_loop(0, ET, unroll=2)
        def _tedge(e):
            asrc = growt[e, pl.ds(DG - 16, 16)]
            adst = arowt[e, pl.ds(0, 16)]
            al = asrc + adst
            al = jnp.maximum(al, 0.2 * al)
            ev = jnp.exp(al)
            for q in range(DG // 16 - 1):
                hq = growt[e, pl.ds(16 * q, 16)]
                eq = _take16(ev, hidx[q])
                msgt[e, pl.ds(16 * q, 16)] = hq * eq
            msgt[e, pl.ds(DG - 16, 16)] = ev * headmask

        pltpu.sync_copy(msgt, acc.at[dstt], add=True)
        plsc.subcore_barrier()

        pltpu.sync_copy(acc.at[pl.ds(rbase, ROWS)],
                        out_hbm.at[cid, pl.ds(rbase, ROWS)])

        @pl.when(sid == 15)
        def _write_tail():
            pltpu.sync_copy(acc.at[pl.ds(16 * ROWS, TAIL)],
                            out_hbm.at[cid, pl.ds(16 * ROWS, TAIL)])

    return edge_pass


_EDGE_PASS_CACHE = {}


def _edge_pass(DG, HD, NH):
    key = (DG, HD, NH)
    if key not in _EDGE_PASS_CACHE:
        _EDGE_PASS_CACHE[key] = _make_edge_pass(DG, HD, NH)
    return _EDGE_PASS_CACHE[key]


def kernel(x, topo, edge_index, W1, att_src1, att_dst1, b1,
           W2, att_src2, att_dst2, b2):
    eye8 = jnp.eye(8, dtype=jnp.float32)
    As1 = (att_src1.reshape(8, 8)[:, :, None] * eye8[:, None, :]).reshape(64, 8)
    Ad1 = (att_dst1.reshape(8, 8)[:, :, None] * eye8[:, None, :]).reshape(64, 8)
    Rep8 = jnp.repeat(eye8, 8, axis=1)
    z15 = jnp.zeros((16, 15), dtype=jnp.float32)
    As2p = jnp.concatenate([att_src2.reshape(16, 1), z15], axis=1)
    Ad2p = jnp.concatenate([att_dst2.reshape(16, 1), z15], axis=1)

    G1, A1 = _tc1(x, topo, W1, As1, Ad1)
    U1 = _edge_pass(80, 8, 8)(G1, edge_index, A1)
    G2, A2 = _tc2(U1, Rep8, b1.reshape(1, 64), W2, As2p, Ad2p)
    U2 = _edge_pass(32, 16, 1)(G2, edge_index, A2)
    return _tc3(U2, b2.reshape(1, 16))
